# baseline (speedup 1.0000x reference)
import functools

import jax
import jax.numpy as jnp
import numpy as np
from jax import lax
from jax.experimental import pallas as pl
from jax.experimental.pallas import tpu as pltpu
from jax.experimental.pallas import tpu_sc as plsc

EMB = 512
HID = 512
NHEADS = 16
NLEVELS = 4
NPOINTS = 8
B = 2
NQ = 1024
LEVEL_SHAPES = [[64, 64], [32, 32], [16, 16], [8, 8]]
I_TOTAL = sum(h * w for h, w in LEVEL_SHAPES)
C = HID // NHEADS
LANES = NHEADS * NLEVELS * NPOINTS
NPTS = B * NQ
NROWS = 4 * LANES
NWORK = 32
PTS_PER_W = NPTS // NWORK

_shapes_np = np.array(LEVEL_SHAPES, np.int32)
_sizes = _shapes_np[:, 0] * _shapes_np[:, 1]
_lev_start = np.concatenate([[0], np.cumsum(_sizes)[:-1]]).astype(np.int32)
_lane_l = (np.arange(LANES) // NPOINTS) % NLEVELS
_lane_h = np.arange(LANES) // (NLEVELS * NPOINTS)
_WM1 = (_shapes_np[_lane_l, 1] - 1).astype(np.float32)
_HM1 = (_shapes_np[_lane_l, 0] - 1).astype(np.float32)
_WVEC = _shapes_np[_lane_l, 1].astype(np.int32)
_LSTART = _lev_start[_lane_l].astype(np.int32)
_HQ544 = ((_lane_h // 2) * 544).astype(np.int32)


def _imgp_body(img_ref, imgn_ref, wi_ref, bi_ref, out_ref):
    x = img_ref[0]
    xn = imgn_ref[0]
    y = jnp.dot(x, wi_ref[...], preferred_element_type=jnp.float32)
    y = y + bi_ref[0][None, :]
    yb = y.astype(jnp.bfloat16)
    yn = jnp.dot(xn, wi_ref[...], preferred_element_type=jnp.float32)
    yn = yn + bi_ref[0][None, :]
    ybn = yn.astype(jnp.bfloat16)
    ysh = jnp.concatenate([yb[1:], ybn[:1]], axis=0)
    IB = yb.shape[0]
    for j in range(NHEADS // 2):
        out_ref[0, pl.ds(j * IB, IB), :] = jnp.concatenate(
            [yb[:, 64 * j:64 * j + 64], ysh[:, 64 * j:64 * j + 64]], axis=1)


IBLK = 544
NBLK = I_TOTAL // IBLK
TROWS = B * NBLK * 8 * IBLK


def _imgp_table(img, Wi, bi):
    grid = (B, NBLK)
    nblk8 = I_TOTAL // 8 - 1
    return pl.pallas_call(
        _imgp_body,
        grid=grid,
        in_specs=[
            pl.BlockSpec((1, IBLK, EMB), lambda b, i: (b, i, 0)),
            pl.BlockSpec((1, 8, EMB),
                         lambda b, i: (b, jnp.minimum((i + 1) * (IBLK // 8), nblk8), 0)),
            pl.BlockSpec((EMB, HID), lambda b, i: (0, 0)),
            pl.BlockSpec((1, HID), lambda b, i: (0, 0)),
        ],
        out_specs=pl.BlockSpec((1, 8 * IBLK, 128), lambda b, i: (b, i, 0)),
        out_shape=jax.ShapeDtypeStruct((B, NBLK * 8 * IBLK, 128), jnp.bfloat16),
    )(img, img, Wi, bi.reshape(1, HID))


def _points_body(q_ref, rpx_ref, rpy_ref, wx_ref, wy_ref, wl_ref, bx_ref,
                 by_ref, bl_ref, wm1_ref, hm1_ref, wvec_ref, lstart_ref,
                 hq_ref, idx_ref, wts_ref):
    b = pl.program_id(0)
    q = q_ref[0]
    rpx_row = rpx_ref[0, 0]
    rpy_row = rpy_ref[0, 0]
    ox = jnp.dot(q, wx_ref[...], preferred_element_type=jnp.float32) + bx_ref[0][None, :]
    oy = jnp.dot(q, wy_ref[...], preferred_element_type=jnp.float32) + by_ref[0][None, :]
    lg = jnp.dot(q, wl_ref[...], preferred_element_type=jnp.float32) + bl_ref[0][None, :]
    QB = ox.shape[0]
    lg3 = lg.reshape(QB, NHEADS, NLEVELS * NPOINTS)
    m = jnp.max(lg3, axis=2, keepdims=True)
    e = jnp.exp(lg3 - m)
    aw = (e / jnp.sum(e, axis=2, keepdims=True)).reshape(QB, LANES)

    wm1 = wm1_ref[0][None, :]
    hm1 = hm1_ref[0][None, :]
    spx = rpx_row[:, None] + ox
    spy = rpy_row[:, None] + oy
    x = jnp.clip(spx * wm1, 0.0, wm1)
    y = jnp.clip(spy * hm1, 0.0, hm1)
    x0f = jnp.minimum(jnp.floor(x), wm1 - 1.0)
    y0f = jnp.minimum(jnp.floor(y), hm1 - 1.0)
    fx = x - x0f
    fy = y - y0f
    x0 = x0f.astype(jnp.int32)
    y0 = y0f.astype(jnp.int32)
    wvec = wvec_ref[0][None, :]
    p0 = lstart_ref[0][None, :] + y0 * wvec + x0
    p1 = p0 + wvec
    bbase = b * (NBLK * 8 * IBLK) + hq_ref[0][None, :]

    def rowid(p):
        pf = (p.astype(jnp.float32) + 0.5) * jnp.float32(1.0 / IBLK)
        blk = jnp.floor(pf).astype(jnp.int32)
        u = p - blk * IBLK
        return bbase + blk * (8 * IBLK) + u

    i0 = rowid(p0)
    i1 = rowid(p1)
    gx = 1.0 - fx
    gy = 1.0 - fy
    w00 = aw * gy * gx
    w01 = aw * gy * fx
    w10 = aw * fy * gx
    w11 = aw * fy * fx
    for t, iv in enumerate((i0, i1)):
        for k in range(4):
            idx_ref[0, :, t * 4 + k, :] = iv[:, k * 128:(k + 1) * 128]
    for t, wv in enumerate((w00, w01, w10, w11)):
        for k in range(4):
            wts_ref[0, :, t * 4 + k, :] = wv[:, k * 128:(k + 1) * 128]


def _points(queries, reference_points, Wq, bq):
    QB = 128
    Wq3 = Wq.reshape(EMB, LANES, 3)
    Wx = Wq3[..., 0]
    Wy = Wq3[..., 1]
    Wl = Wq3[..., 2]
    bq3 = bq.reshape(LANES, 3)
    bx = bq3[:, 0].reshape(1, LANES)
    by = bq3[:, 1].reshape(1, LANES)
    bl = bq3[:, 2].reshape(1, LANES)
    rpx = reference_points[..., 0].reshape(B * (NQ // QB), 1, QB)
    rpy = reference_points[..., 1].reshape(B * (NQ // QB), 1, QB)
    grid = (B, NQ // QB)
    full = lambda b, i: (0, 0)
    idx, wts = pl.pallas_call(
        _points_body,
        grid=grid,
        in_specs=[
            pl.BlockSpec((1, QB, EMB), lambda b, i: (b, i, 0)),
            pl.BlockSpec((1, 1, QB), lambda b, i: (b * (NQ // QB) + i, 0, 0)),
            pl.BlockSpec((1, 1, QB), lambda b, i: (b * (NQ // QB) + i, 0, 0)),
            pl.BlockSpec((EMB, LANES), full),
            pl.BlockSpec((EMB, LANES), full),
            pl.BlockSpec((EMB, LANES), full),
            pl.BlockSpec((1, LANES), full),
            pl.BlockSpec((1, LANES), full),
            pl.BlockSpec((1, LANES), full),
            pl.BlockSpec((1, LANES), full),
            pl.BlockSpec((1, LANES), full),
            pl.BlockSpec((1, LANES), full),
            pl.BlockSpec((1, LANES), full),
            pl.BlockSpec((1, LANES), full),
        ],
        out_specs=[
            pl.BlockSpec((1, QB, 8, 128), lambda b, i: (b, i, 0, 0)),
            pl.BlockSpec((1, QB, 16, 128), lambda b, i: (b, i, 0, 0)),
        ],
        out_shape=[
            jax.ShapeDtypeStruct((B, NQ, 8, 128), jnp.int32),
            jax.ShapeDtypeStruct((B, NQ, 16, 128), jnp.float32),
        ],
    )(queries, rpx, rpy, Wx, Wy, Wl, bx, by, bl,
      jnp.asarray(_WM1).reshape(1, LANES), jnp.asarray(_HM1).reshape(1, LANES),
      jnp.asarray(_WVEC).reshape(1, LANES), jnp.asarray(_LSTART).reshape(1, LANES),
      jnp.asarray(_HQ544).reshape(1, LANES))
    return idx.reshape(NPTS, 8, 128), wts.reshape(NPTS, 16, 128)


def _sc_body(table_hbm, idx_hbm, wts_hbm, out_hbm, idxv, wtsv, rowsv, outv,
             sem_rows0, sem_rows1, sem_idx, sem_wts0, sem_wts1,
             sem_out0, sem_out1):
    wid = lax.axis_index("s") * 2 + lax.axis_index("c")
    base = wid * PTS_PER_W
    sem_rows = (sem_rows0, sem_rows1)
    sem_wts = (sem_wts0, sem_wts1)
    sem_out = (sem_out0, sem_out1)

    def clamp(pt):
        return jnp.minimum(pt, NPTS - 1)

    def idx_copy(pt, t, s):
        return pltpu.make_async_copy(
            idx_hbm.at[clamp(pt), pl.ds(t * 4, 4)], idxv.at[s], sem_idx)

    def wts_copy(pt, t, s):
        return pltpu.make_async_copy(
            wts_hbm.at[clamp(pt), pl.ds(t * 8, 8)], wtsv.at[s], sem_wts[s])

    def gathers(s):
        return [
            pltpu.make_async_copy(
                table_hbm.at[idxv.at[s, j]],
                rowsv.at[s, pl.ds(j * 128, 128), :],
                sem_rows[s],
            )
            for j in range(4)
        ]

    def start(cs):
        for cp in cs:
            cp.start()

    def wait(cs):
        for cp in cs:
            cp.wait()

    def out_copy(pt, po):
        return pltpu.make_async_copy(outv.at[po], out_hbm.at[clamp(pt)], sem_out[po])

    def compute_half(pt, s, po, t, k):
        if t == 0:
            @pl.when(k > 0)
            def _():
                out_copy(pt - 2, po).wait()

        def h_body(h, _):
            off = (h % 2) * C
            acc = [jnp.zeros((16,), jnp.float32) for _ in range(4)]
            for g in range(2):
                jb = h * (NLEVELS * NPOINTS) + g * 16
                wrow = jb // 128
                wcol = jb % 128
                wlv = wtsv[s, wrow, pl.ds(wcol, 16)]
                wrv = wtsv[s, 4 + wrow, pl.ds(wcol, 16)]
                for q in range(4):
                    p = []
                    for e in range(4):
                        j = q * 4 + e
                        wl = wlv[j]
                        wr = wrv[j]
                        wsl = jnp.full((16,), wl, jnp.float32)
                        wbl = plsc.pack(wsl, wsl,
                                        format=plsc.PackFormat.INTERLEAVED)
                        wsr = jnp.full((16,), wr, jnp.float32)
                        wbr = plsc.pack(wsr, wsr,
                                        format=plsc.PackFormat.INTERLEAVED)
                        seg = jb + j
                        p.append(rowsv[s, seg, pl.ds(off, 32)] * wbl
                                 + rowsv[s, seg, pl.ds(off + 2 * C, 32)] * wbr)
                    tree = (p[0] + p[1]) + (p[2] + p[3])
                    rev, rod = plsc.unpack(
                        tree, format=plsc.PackFormat.INTERLEAVED)
                    kk = (q & 1) * 2
                    acc[kk] = acc[kk] + rev
                    acc[kk + 1] = acc[kk + 1] + rod
            lo = acc[0] + acc[2]
            hi = acc[1] + acc[3]
            if t == 0:
                outv[po, pl.ds(h * C, 16)] = lo
                outv[po, pl.ds(h * C + 16, 16)] = hi
            else:
                outv[po, pl.ds(h * C, 16)] = outv[po, pl.ds(h * C, 16)] + lo
                outv[po, pl.ds(h * C + 16, 16)] = (
                    outv[po, pl.ds(h * C + 16, 16)] + hi)
            return 0

        lax.fori_loop(0, NHEADS, h_body, 0)
        if t == 1:
            out_copy(pt, po).start()

    idx_copy(base, 0, 0).start()
    wts_copy(base, 0, 0).start()
    wts_copy(base, 1, 1).start()
    idx_copy(base, 0, 0).wait()
    start(gathers(0))
    idx_copy(base, 1, 1).start()

    def half_step(cur, nxt, nxt2, k):
        pt, t, s, po = cur
        pt1, t1, s1 = nxt
        pt2, t2 = nxt2
        idx_copy(pt1, t1, s1).wait()
        start(gathers(s1))
        wait(gathers(s))
        idx_copy(pt2, t2, s).start()
        wts_copy(pt, t, s).wait()
        compute_half(pt, s, po, t, k)
        wts_copy(pt2, t2, s).start()

    def pair_body(k, _):
        a = base + 2 * k
        half_step((a, 0, 0, 0), (a, 1, 1), (a + 1, 0), k)
        half_step((a, 1, 1, 0), (a + 1, 0, 0), (a + 1, 1), k)
        half_step((a + 1, 0, 0, 1), (a + 1, 1, 1), (a + 2, 0), k)
        half_step((a + 1, 1, 1, 1), (a + 2, 0, 0), (a + 2, 1), k)
        return 0

    lax.fori_loop(0, PTS_PER_W // 2, pair_body, 0)
    last = base + PTS_PER_W - 1
    wait(gathers(0))
    idx_copy(0, 1, 1).wait()
    wts_copy(0, 0, 0).wait()
    wts_copy(0, 1, 1).wait()
    out_copy(last - 1, 0).wait()
    out_copy(last, 1).wait()


def _sc_gather(table, idx, wts):
    mesh = plsc.VectorSubcoreMesh(core_axis_name="c", subcore_axis_name="s")
    f = functools.partial(
        pl.kernel,
        mesh=mesh,
        compiler_params=pltpu.CompilerParams(use_tc_tiling_on_sc=False,
                                             needs_layout_passes=False),
        out_type=jax.ShapeDtypeStruct((NPTS, HID), jnp.float32),
        scratch_types=[
            pltpu.VMEM((2, 4, 128), jnp.int32),
            pltpu.VMEM((2, 8, 128), jnp.float32),
            pltpu.VMEM((2, LANES, 128), jnp.bfloat16),
            pltpu.VMEM((2, HID), jnp.float32),
            pltpu.SemaphoreType.DMA,
            pltpu.SemaphoreType.DMA,
            pltpu.SemaphoreType.DMA,
            pltpu.SemaphoreType.DMA,
            pltpu.SemaphoreType.DMA,
            pltpu.SemaphoreType.DMA,
            pltpu.SemaphoreType.DMA,
        ],
    )(_sc_body)
    return f(table.reshape(TROWS, 128), idx, wts)


def _proj_body(x_ref, wo_ref, bo_ref, out_ref):
    out_ref[...] = (jnp.dot(x_ref[...], wo_ref[...],
                            preferred_element_type=jnp.float32)
                    + bo_ref[0][None, :])


_kk = np.tile(np.arange(C), NHEADS)
_hh = np.repeat(np.arange(NHEADS), C) * C
_PERM = (_hh + np.where(_kk < 16, 2 * _kk, 2 * (_kk - 16) + 1)).astype(np.int32)


def _out_proj(acc, Wo, bo):
    MB = 256
    return pl.pallas_call(
        _proj_body,
        grid=(NPTS // MB,),
        in_specs=[
            pl.BlockSpec((MB, HID), lambda i: (i, 0)),
            pl.BlockSpec((HID, EMB), lambda i: (0, 0)),
            pl.BlockSpec((1, EMB), lambda i: (0, 0)),
        ],
        out_specs=pl.BlockSpec((MB, EMB), lambda i: (i, 0)),
        out_shape=jax.ShapeDtypeStruct((NPTS, EMB), jnp.float32),
    )(acc, Wo[jnp.asarray(_PERM)], bo.reshape(1, EMB))


def kernel(img, shapes, queries, reference_points, Wi, bi, Wq, bq, Wo, bo):
    table = _imgp_table(img, Wi, bi)
    idx, wts = _points(queries, reference_points, Wq, bq)
    acc = _sc_gather(table, idx, wts)
    out = _out_proj(acc, Wo, bo)
    return out.reshape(B, NQ, EMB)


# speedup vs baseline: 1.2011x; 1.2011x over previous
import functools

import jax
import jax.numpy as jnp
import numpy as np
from jax import lax
from jax.experimental import pallas as pl
from jax.experimental.pallas import tpu as pltpu
from jax.experimental.pallas import tpu_sc as plsc

EMB = 512
HID = 512
NHEADS = 16
NLEVELS = 4
NPOINTS = 8
B = 2
NQ = 1024
LEVEL_SHAPES = [[64, 64], [32, 32], [16, 16], [8, 8]]
I_TOTAL = sum(h * w for h, w in LEVEL_SHAPES)
C = HID // NHEADS
LANES = NHEADS * NLEVELS * NPOINTS
NPTS = B * NQ
NROWS = 4 * LANES
NWORK = 32
PTS_PER_W = NPTS // NWORK

_shapes_np = np.array(LEVEL_SHAPES, np.int32)
_sizes = _shapes_np[:, 0] * _shapes_np[:, 1]
_lev_start = np.concatenate([[0], np.cumsum(_sizes)[:-1]]).astype(np.int32)
_lane_l = (np.arange(LANES) // NPOINTS) % NLEVELS
_lane_h = np.arange(LANES) // (NLEVELS * NPOINTS)
_WM1 = (_shapes_np[_lane_l, 1] - 1).astype(np.float32)
_HM1 = (_shapes_np[_lane_l, 0] - 1).astype(np.float32)
_WVEC = _shapes_np[_lane_l, 1].astype(np.int32)
_LSTART = _lev_start[_lane_l].astype(np.int32)
IBLK = 544
NBLK = I_TOTAL // IBLK
_H544 = (_lane_h * IBLK).astype(np.int32)


def _imgp_body(img_ref, imgn_ref, wi_ref, bi_ref, out_ref):
    x = img_ref[0]
    xn = imgn_ref[0]
    y = jnp.dot(x, wi_ref[...], preferred_element_type=jnp.float32)
    y = y + bi_ref[0][None, :]
    yb = y.astype(jnp.bfloat16)
    yn = jnp.dot(xn, wi_ref[...], preferred_element_type=jnp.float32)
    yn = yn + bi_ref[0][None, :]
    ybn = yn.astype(jnp.bfloat16)
    ysh = jnp.concatenate([yb[1:], ybn[:1]], axis=0)
    IB = yb.shape[0]
    for h in range(NHEADS):
        out_ref[0, pl.ds(h * IB, IB), :] = jnp.concatenate(
            [yb[:, h * C:(h + 1) * C], ysh[:, h * C:(h + 1) * C]], axis=1)


def _imgp_table(img, Wi, bi):
    grid = (B, NBLK)
    nblk8 = I_TOTAL // 8 - 1
    return pl.pallas_call(
        _imgp_body,
        grid=grid,
        in_specs=[
            pl.BlockSpec((1, IBLK, EMB), lambda b, i: (b, i, 0)),
            pl.BlockSpec((1, 8, EMB),
                         lambda b, i: (b, jnp.minimum((i + 1) * (IBLK // 8), nblk8), 0)),
            pl.BlockSpec((EMB, HID), lambda b, i: (0, 0)),
            pl.BlockSpec((1, HID), lambda b, i: (0, 0)),
        ],
        out_specs=pl.BlockSpec((1, NHEADS * IBLK, 2 * C), lambda b, i: (b, i, 0)),
        out_shape=jax.ShapeDtypeStruct((B, NBLK * NHEADS * IBLK, 2 * C),
                                       jnp.bfloat16),
    )(img, img, Wi, bi.reshape(1, HID))


def _points_body(q_ref, rpx_ref, rpy_ref, wx_ref, wy_ref, wl_ref, bx_ref,
                 by_ref, bl_ref, wm1_ref, hm1_ref, wvec_ref, lstart_ref,
                 h544_ref, idx_ref, wts_ref):
    b = pl.program_id(0)
    q = q_ref[0]
    rpx_row = rpx_ref[0, 0]
    rpy_row = rpy_ref[0, 0]
    ox = jnp.dot(q, wx_ref[...], preferred_element_type=jnp.float32) + bx_ref[0][None, :]
    oy = jnp.dot(q, wy_ref[...], preferred_element_type=jnp.float32) + by_ref[0][None, :]
    lg = jnp.dot(q, wl_ref[...], preferred_element_type=jnp.float32) + bl_ref[0][None, :]
    QB = ox.shape[0]
    lg3 = lg.reshape(QB, NHEADS, NLEVELS * NPOINTS)
    m = jnp.max(lg3, axis=2, keepdims=True)
    e = jnp.exp(lg3 - m)
    aw = (e / jnp.sum(e, axis=2, keepdims=True)).reshape(QB, LANES)

    wm1 = wm1_ref[0][None, :]
    hm1 = hm1_ref[0][None, :]
    spx = rpx_row[:, None] + ox
    spy = rpy_row[:, None] + oy
    x = jnp.clip(spx * wm1, 0.0, wm1)
    y = jnp.clip(spy * hm1, 0.0, hm1)
    x0f = jnp.minimum(jnp.floor(x), wm1 - 1.0)
    y0f = jnp.minimum(jnp.floor(y), hm1 - 1.0)
    fx = x - x0f
    fy = y - y0f
    x0 = x0f.astype(jnp.int32)
    y0 = y0f.astype(jnp.int32)
    wvec = wvec_ref[0][None, :]
    p0 = lstart_ref[0][None, :] + y0 * wvec + x0
    p1 = p0 + wvec
    bbase = b * (NBLK * NHEADS * IBLK) + h544_ref[0][None, :]

    def rowid(p):
        pf = (p.astype(jnp.float32) + 0.5) * jnp.float32(1.0 / IBLK)
        blk = jnp.floor(pf).astype(jnp.int32)
        u = p - blk * IBLK
        return bbase + blk * (NHEADS * IBLK) + u

    i0 = rowid(p0)
    i1 = rowid(p1)
    gx = 1.0 - fx
    gy = 1.0 - fy
    w00 = aw * gy * gx
    w01 = aw * gy * fx
    w10 = aw * fy * gx
    w11 = aw * fy * fx
    for t, iv in enumerate((i0, i1)):
        for k in range(4):
            idx_ref[0, :, t * 4 + k, :] = iv[:, k * 128:(k + 1) * 128]
    for t, wv in enumerate((w00, w01, w10, w11)):
        wts_ref[0, :, pl.ds(t * LANES, LANES)] = wv


def _points(queries, reference_points, Wq, bq):
    QB = 128
    Wq3 = Wq.reshape(EMB, LANES, 3)
    Wx = Wq3[..., 0]
    Wy = Wq3[..., 1]
    Wl = Wq3[..., 2]
    bq3 = bq.reshape(LANES, 3)
    bx = bq3[:, 0].reshape(1, LANES)
    by = bq3[:, 1].reshape(1, LANES)
    bl = bq3[:, 2].reshape(1, LANES)
    rpx = reference_points[..., 0].reshape(B * (NQ // QB), 1, QB)
    rpy = reference_points[..., 1].reshape(B * (NQ // QB), 1, QB)
    grid = (B, NQ // QB)
    full = lambda b, i: (0, 0)
    idx, wts = pl.pallas_call(
        _points_body,
        grid=grid,
        in_specs=[
            pl.BlockSpec((1, QB, EMB), lambda b, i: (b, i, 0)),
            pl.BlockSpec((1, 1, QB), lambda b, i: (b * (NQ // QB) + i, 0, 0)),
            pl.BlockSpec((1, 1, QB), lambda b, i: (b * (NQ // QB) + i, 0, 0)),
            pl.BlockSpec((EMB, LANES), full),
            pl.BlockSpec((EMB, LANES), full),
            pl.BlockSpec((EMB, LANES), full),
            pl.BlockSpec((1, LANES), full),
            pl.BlockSpec((1, LANES), full),
            pl.BlockSpec((1, LANES), full),
            pl.BlockSpec((1, LANES), full),
            pl.BlockSpec((1, LANES), full),
            pl.BlockSpec((1, LANES), full),
            pl.BlockSpec((1, LANES), full),
            pl.BlockSpec((1, LANES), full),
        ],
        out_specs=[
            pl.BlockSpec((1, QB, 8, 128), lambda b, i: (b, i, 0, 0)),
            pl.BlockSpec((1, QB, NROWS), lambda b, i: (b, i, 0)),
        ],
        out_shape=[
            jax.ShapeDtypeStruct((B, NQ, 8, 128), jnp.int32),
            jax.ShapeDtypeStruct((B, NQ, NROWS), jnp.float32),
        ],
    )(queries, rpx, rpy, Wx, Wy, Wl, bx, by, bl,
      jnp.asarray(_WM1).reshape(1, LANES), jnp.asarray(_HM1).reshape(1, LANES),
      jnp.asarray(_WVEC).reshape(1, LANES), jnp.asarray(_LSTART).reshape(1, LANES),
      jnp.asarray(_H544).reshape(1, LANES))
    return idx.reshape(NPTS, 8, 128), wts.reshape(NPTS, NROWS)


def _sc_body(table_hbm, idx_hbm, wts_hbm, out_hbm, idxv, wtsv, rowsv, outv,
             sem_rows0, sem_rows1, sem_idx, sem_wts0, sem_wts1,
             sem_out0, sem_out1):
    wid = lax.axis_index("s") * 2 + lax.axis_index("c")
    base = wid * PTS_PER_W
    sem_rows = (sem_rows0, sem_rows1)
    sem_wts = (sem_wts0, sem_wts1)
    sem_out = (sem_out0, sem_out1)

    def clamp(pt):
        return jnp.minimum(pt, NPTS - 1)

    def idx_copy(pt, s):
        return pltpu.make_async_copy(idx_hbm.at[clamp(pt)], idxv.at[s], sem_idx)

    def wts_copy(pt, s):
        return pltpu.make_async_copy(wts_hbm.at[clamp(pt)], wtsv.at[s], sem_wts[s])

    def gathers(s):
        return [
            pltpu.make_async_copy(
                table_hbm.at[idxv.at[s, j]],
                rowsv.at[s, pl.ds(j * 128, 128), :],
                sem_rows[s],
            )
            for j in range(8)
        ]

    def start(cs):
        for cp in cs:
            cp.start()

    def wait(cs):
        for cp in cs:
            cp.wait()

    def out_copy(pt, s):
        return pltpu.make_async_copy(outv.at[s], out_hbm.at[clamp(pt)], sem_out[s])

    def compute(pt, s, k):
        @pl.when(k > 0)
        def _():
            out_copy(pt - 2, s).wait()

        def h_body(h, _):
            hb = h * (NLEVELS * NPOINTS)
            acc = [jnp.zeros((16,), jnp.float32) for _ in range(4)]
            for t in range(2):
                for g in range(2):
                    sb = t * LANES + hb + g * 16
                    wlv = wtsv[s, pl.ds(2 * t * LANES + hb + g * 16, 16)]
                    wrv = wtsv[s, pl.ds((2 * t + 1) * LANES + hb + g * 16, 16)]
                    for q in range(4):
                        p = []
                        for e in range(4):
                            j = q * 4 + e
                            wl = wlv[j]
                            wr = wrv[j]
                            wsl = jnp.full((16,), wl, jnp.float32)
                            wbl = plsc.pack(wsl, wsl,
                                            format=plsc.PackFormat.INTERLEAVED)
                            wsr = jnp.full((16,), wr, jnp.float32)
                            wbr = plsc.pack(wsr, wsr,
                                            format=plsc.PackFormat.INTERLEAVED)
                            p.append(rowsv[s, sb + j, pl.ds(0, 32)] * wbl
                                     + rowsv[s, sb + j, pl.ds(32, 32)] * wbr)
                        tree = (p[0] + p[1]) + (p[2] + p[3])
                        rev, rod = plsc.unpack(
                            tree, format=plsc.PackFormat.INTERLEAVED)
                        kk = (q & 1) * 2
                        acc[kk] = acc[kk] + rev
                        acc[kk + 1] = acc[kk + 1] + rod
            outv[s, pl.ds(h * C, 16)] = acc[0] + acc[2]
            outv[s, pl.ds(h * C + 16, 16)] = acc[1] + acc[3]
            return 0

        lax.fori_loop(0, NHEADS, h_body, 0)
        out_copy(pt, s).start()

    idx_copy(base, 0).start()
    wts_copy(base, 0).start()
    wts_copy(base + 1, 1).start()
    idx_copy(base, 0).wait()
    start(gathers(0))
    idx_copy(base + 1, 1).start()

    def pair_body(k, _):
        a = base + 2 * k
        idx_copy(a + 1, 1).wait()
        start(gathers(1))
        wait(gathers(0))
        idx_copy(a + 2, 0).start()
        wts_copy(a, 0).wait()
        compute(a, 0, k)
        wts_copy(a + 2, 0).start()
        idx_copy(a + 2, 0).wait()
        start(gathers(0))
        wait(gathers(1))
        idx_copy(a + 3, 1).start()
        wts_copy(a + 1, 1).wait()
        compute(a + 1, 1, k)
        wts_copy(a + 3, 1).start()
        return 0

    lax.fori_loop(0, PTS_PER_W // 2, pair_body, 0)
    last = base + PTS_PER_W - 1
    wait(gathers(0))
    idx_copy(last + 2, 1).wait()
    wts_copy(last + 1, 0).wait()
    wts_copy(last + 2, 1).wait()
    out_copy(last - 1, 0).wait()
    out_copy(last, 1).wait()


def _sc_gather(table, idx, wts):
    mesh = plsc.VectorSubcoreMesh(core_axis_name="c", subcore_axis_name="s")
    f = functools.partial(
        pl.kernel,
        mesh=mesh,
        compiler_params=pltpu.CompilerParams(use_tc_tiling_on_sc=False,
                                             needs_layout_passes=False),
        out_type=jax.ShapeDtypeStruct((NPTS, HID), jnp.float32),
        scratch_types=[
            pltpu.VMEM((2, 8, 128), jnp.int32),
            pltpu.VMEM((2, NROWS), jnp.float32),
            pltpu.VMEM((2, NROWS // 2, 2 * C), jnp.bfloat16),
            pltpu.VMEM((2, HID), jnp.float32),
            pltpu.SemaphoreType.DMA,
            pltpu.SemaphoreType.DMA,
            pltpu.SemaphoreType.DMA,
            pltpu.SemaphoreType.DMA,
            pltpu.SemaphoreType.DMA,
            pltpu.SemaphoreType.DMA,
            pltpu.SemaphoreType.DMA,
        ],
    )(_sc_body)
    return f(table.reshape(B * NBLK * NHEADS * IBLK, 2 * C), idx, wts)


def _proj_body(x_ref, wo_ref, bo_ref, out_ref):
    out_ref[...] = (jnp.dot(x_ref[...], wo_ref[...],
                            preferred_element_type=jnp.float32)
                    + bo_ref[0][None, :])


_kk = np.tile(np.arange(C), NHEADS)
_hh = np.repeat(np.arange(NHEADS), C) * C
_PERM = (_hh + np.where(_kk < 16, 2 * _kk, 2 * (_kk - 16) + 1)).astype(np.int32)


def _out_proj(acc, Wo, bo):
    MB = 256
    return pl.pallas_call(
        _proj_body,
        grid=(NPTS // MB,),
        in_specs=[
            pl.BlockSpec((MB, HID), lambda i: (i, 0)),
            pl.BlockSpec((HID, EMB), lambda i: (0, 0)),
            pl.BlockSpec((1, EMB), lambda i: (0, 0)),
        ],
        out_specs=pl.BlockSpec((MB, EMB), lambda i: (i, 0)),
        out_shape=jax.ShapeDtypeStruct((NPTS, EMB), jnp.float32),
    )(acc, Wo[jnp.asarray(_PERM)], bo.reshape(1, EMB))


def kernel(img, shapes, queries, reference_points, Wi, bi, Wq, bq, Wo, bo):
    table = _imgp_table(img, Wi, bi)
    idx, wts = _points(queries, reference_points, Wq, bq)
    acc = _sc_gather(table, idx, wts)
    out = _out_proj(acc, Wo, bo)
    return out.reshape(B, NQ, EMB)


# speedup vs baseline: 1.2015x; 1.0003x over previous
import functools

import jax
import jax.numpy as jnp
import numpy as np
from jax import lax
from jax.experimental import pallas as pl
from jax.experimental.pallas import tpu as pltpu
from jax.experimental.pallas import tpu_sc as plsc

EMB = 512
HID = 512
NHEADS = 16
NLEVELS = 4
NPOINTS = 8
B = 2
NQ = 1024
LEVEL_SHAPES = [[64, 64], [32, 32], [16, 16], [8, 8]]
I_TOTAL = sum(h * w for h, w in LEVEL_SHAPES)
C = HID // NHEADS
LANES = NHEADS * NLEVELS * NPOINTS
NPTS = B * NQ
NROWS = 4 * LANES
NWORK = 32
PTS_PER_W = NPTS // NWORK

_shapes_np = np.array(LEVEL_SHAPES, np.int32)
_sizes = _shapes_np[:, 0] * _shapes_np[:, 1]
_lev_start = np.concatenate([[0], np.cumsum(_sizes)[:-1]]).astype(np.int32)
_lane_l = (np.arange(LANES) // NPOINTS) % NLEVELS
_lane_h = np.arange(LANES) // (NLEVELS * NPOINTS)
_WM1 = (_shapes_np[_lane_l, 1] - 1).astype(np.float32)
_HM1 = (_shapes_np[_lane_l, 0] - 1).astype(np.float32)
_WVEC = _shapes_np[_lane_l, 1].astype(np.int32)
_LSTART = _lev_start[_lane_l].astype(np.int32)
IBLK = 544
NBLK = I_TOTAL // IBLK
_H544 = (_lane_h * IBLK).astype(np.int32)


def _imgp_body(img_ref, imgn_ref, wi_ref, bi_ref, out_ref):
    x = img_ref[0]
    xn = imgn_ref[0]
    y = jnp.dot(x, wi_ref[...], preferred_element_type=jnp.float32)
    y = y + bi_ref[0][None, :]
    yb = y.astype(jnp.bfloat16)
    yn = jnp.dot(xn, wi_ref[...], preferred_element_type=jnp.float32)
    yn = yn + bi_ref[0][None, :]
    ybn = yn.astype(jnp.bfloat16)
    ysh = jnp.concatenate([yb[1:], ybn[:1]], axis=0)
    IB = yb.shape[0]
    for h in range(NHEADS):
        out_ref[pl.ds(h * IB, IB), :] = jnp.concatenate(
            [yb[:, h * C:(h + 1) * C], ysh[:, h * C:(h + 1) * C]], axis=1)


def _imgp_table(img, Wi, bi):
    grid = (B, NBLK)
    nblk8 = I_TOTAL // 8 - 1
    return pl.pallas_call(
        _imgp_body,
        grid=grid,
        in_specs=[
            pl.BlockSpec((1, IBLK, EMB), lambda b, i: (b, i, 0)),
            pl.BlockSpec((1, 8, EMB),
                         lambda b, i: (b, jnp.minimum((i + 1) * (IBLK // 8), nblk8), 0)),
            pl.BlockSpec((EMB, HID), lambda b, i: (0, 0)),
            pl.BlockSpec((1, HID), lambda b, i: (0, 0)),
        ],
        out_specs=pl.BlockSpec((NHEADS * IBLK, 2 * C),
                               lambda b, i: (b * NBLK + i, 0)),
        out_shape=jax.ShapeDtypeStruct((B * NBLK * NHEADS * IBLK, 2 * C),
                                       jnp.bfloat16),
    )(img, img, Wi, bi.reshape(1, HID))


def _points_body(q_ref, rpx_ref, rpy_ref, wx_ref, wy_ref, wl_ref, bx_ref,
                 by_ref, bl_ref, wm1_ref, hm1_ref, wvec_ref, lstart_ref,
                 h544_ref, idx_ref, wts_ref):
    b = pl.program_id(0)
    q = q_ref[0]
    rpx_row = rpx_ref[0, 0]
    rpy_row = rpy_ref[0, 0]
    ox = jnp.dot(q, wx_ref[...], preferred_element_type=jnp.float32) + bx_ref[0][None, :]
    oy = jnp.dot(q, wy_ref[...], preferred_element_type=jnp.float32) + by_ref[0][None, :]
    lg = jnp.dot(q, wl_ref[...], preferred_element_type=jnp.float32) + bl_ref[0][None, :]
    QB = ox.shape[0]
    lg3 = lg.reshape(QB, NHEADS, NLEVELS * NPOINTS)
    m = jnp.max(lg3, axis=2, keepdims=True)
    e = jnp.exp(lg3 - m)
    aw = (e / jnp.sum(e, axis=2, keepdims=True)).reshape(QB, LANES)

    wm1 = wm1_ref[0][None, :]
    hm1 = hm1_ref[0][None, :]
    spx = rpx_row[:, None] + ox
    spy = rpy_row[:, None] + oy
    x = jnp.clip(spx * wm1, 0.0, wm1)
    y = jnp.clip(spy * hm1, 0.0, hm1)
    x0f = jnp.minimum(jnp.floor(x), wm1 - 1.0)
    y0f = jnp.minimum(jnp.floor(y), hm1 - 1.0)
    fx = x - x0f
    fy = y - y0f
    x0 = x0f.astype(jnp.int32)
    y0 = y0f.astype(jnp.int32)
    wvec = wvec_ref[0][None, :]
    p0 = lstart_ref[0][None, :] + y0 * wvec + x0
    p1 = p0 + wvec
    bbase = b * (NBLK * NHEADS * IBLK) + h544_ref[0][None, :]

    def rowid(p):
        pf = (p.astype(jnp.float32) + 0.5) * jnp.float32(1.0 / IBLK)
        blk = jnp.floor(pf).astype(jnp.int32)
        u = p - blk * IBLK
        return bbase + blk * (NHEADS * IBLK) + u

    i0 = rowid(p0)
    i1 = rowid(p1)
    gx = 1.0 - fx
    gy = 1.0 - fy
    w00 = aw * gy * gx
    w01 = aw * gy * fx
    w10 = aw * fy * gx
    w11 = aw * fy * fx
    for t, iv in enumerate((i0, i1)):
        for k in range(4):
            idx_ref[0, :, t * 4 + k, :] = iv[:, k * 128:(k + 1) * 128]
    for t, wv in enumerate((w00, w01, w10, w11)):
        wts_ref[0, :, pl.ds(t * LANES, LANES)] = wv


def _points(queries, reference_points, Wq, bq):
    QB = 128
    Wq3 = Wq.reshape(EMB, LANES, 3)
    Wx = Wq3[..., 0]
    Wy = Wq3[..., 1]
    Wl = Wq3[..., 2]
    bq3 = bq.reshape(LANES, 3)
    bx = bq3[:, 0].reshape(1, LANES)
    by = bq3[:, 1].reshape(1, LANES)
    bl = bq3[:, 2].reshape(1, LANES)
    rpx = reference_points[..., 0].reshape(B * (NQ // QB), 1, QB)
    rpy = reference_points[..., 1].reshape(B * (NQ // QB), 1, QB)
    grid = (B, NQ // QB)
    full = lambda b, i: (0, 0)
    idx, wts = pl.pallas_call(
        _points_body,
        grid=grid,
        in_specs=[
            pl.BlockSpec((1, QB, EMB), lambda b, i: (b, i, 0)),
            pl.BlockSpec((1, 1, QB), lambda b, i: (b * (NQ // QB) + i, 0, 0)),
            pl.BlockSpec((1, 1, QB), lambda b, i: (b * (NQ // QB) + i, 0, 0)),
            pl.BlockSpec((EMB, LANES), full),
            pl.BlockSpec((EMB, LANES), full),
            pl.BlockSpec((EMB, LANES), full),
            pl.BlockSpec((1, LANES), full),
            pl.BlockSpec((1, LANES), full),
            pl.BlockSpec((1, LANES), full),
            pl.BlockSpec((1, LANES), full),
            pl.BlockSpec((1, LANES), full),
            pl.BlockSpec((1, LANES), full),
            pl.BlockSpec((1, LANES), full),
            pl.BlockSpec((1, LANES), full),
        ],
        out_specs=[
            pl.BlockSpec((1, QB, 8, 128), lambda b, i: (b, i, 0, 0)),
            pl.BlockSpec((1, QB, NROWS), lambda b, i: (b, i, 0)),
        ],
        out_shape=[
            jax.ShapeDtypeStruct((B, NQ, 8, 128), jnp.int32),
            jax.ShapeDtypeStruct((B, NQ, NROWS), jnp.float32),
        ],
    )(queries, rpx, rpy, Wx, Wy, Wl, bx, by, bl,
      jnp.asarray(_WM1).reshape(1, LANES), jnp.asarray(_HM1).reshape(1, LANES),
      jnp.asarray(_WVEC).reshape(1, LANES), jnp.asarray(_LSTART).reshape(1, LANES),
      jnp.asarray(_H544).reshape(1, LANES))
    return idx.reshape(NPTS, 8, 128), wts.reshape(NPTS, NROWS)


def _sc_body(table_hbm, idx_hbm, wts_hbm, out_hbm, idxv, wtsv, rowsv, outv,
             sem_rows0, sem_rows1, sem_idx, sem_wts0, sem_wts1,
             sem_out0, sem_out1):
    wid = lax.axis_index("s") * 2 + lax.axis_index("c")
    base = wid * PTS_PER_W
    sem_rows = (sem_rows0, sem_rows1)
    sem_wts = (sem_wts0, sem_wts1)
    sem_out = (sem_out0, sem_out1)

    def clamp(pt):
        return jnp.minimum(pt, NPTS - 1)

    def idx_copy(pt, s):
        return pltpu.make_async_copy(idx_hbm.at[clamp(pt)], idxv.at[s], sem_idx)

    def wts_copy(pt, s):
        return pltpu.make_async_copy(wts_hbm.at[clamp(pt)], wtsv.at[s], sem_wts[s])

    def gathers(s):
        return [
            pltpu.make_async_copy(
                table_hbm.at[idxv.at[s, j]],
                rowsv.at[s, pl.ds(j * 128, 128), :],
                sem_rows[s],
            )
            for j in range(8)
        ]

    def start(cs):
        for cp in cs:
            cp.start()

    def wait(cs):
        for cp in cs:
            cp.wait()

    def out_copy(pt, s):
        return pltpu.make_async_copy(outv.at[s], out_hbm.at[clamp(pt)], sem_out[s])

    def compute(pt, s, k):
        @pl.when(k > 0)
        def _():
            out_copy(pt - 2, s).wait()

        def h_body(h, _):
            hb = h * (NLEVELS * NPOINTS)
            acc = [jnp.zeros((16,), jnp.float32) for _ in range(4)]
            for t in range(2):
                for g in range(2):
                    sb = t * LANES + hb + g * 16
                    wlv = wtsv[s, pl.ds(2 * t * LANES + hb + g * 16, 16)]
                    wrv = wtsv[s, pl.ds((2 * t + 1) * LANES + hb + g * 16, 16)]
                    for q in range(4):
                        p = []
                        for e in range(4):
                            j = q * 4 + e
                            wl = wlv[j]
                            wr = wrv[j]
                            wsl = jnp.full((16,), wl, jnp.float32)
                            wbl = plsc.pack(wsl, wsl,
                                            format=plsc.PackFormat.INTERLEAVED)
                            wsr = jnp.full((16,), wr, jnp.float32)
                            wbr = plsc.pack(wsr, wsr,
                                            format=plsc.PackFormat.INTERLEAVED)
                            p.append(rowsv[s, sb + j, pl.ds(0, 32)] * wbl
                                     + rowsv[s, sb + j, pl.ds(32, 32)] * wbr)
                        tree = (p[0] + p[1]) + (p[2] + p[3])
                        rev, rod = plsc.unpack(
                            tree, format=plsc.PackFormat.INTERLEAVED)
                        kk = (q & 1) * 2
                        acc[kk] = acc[kk] + rev
                        acc[kk + 1] = acc[kk + 1] + rod
            outv[s, pl.ds(h * C, 16)] = acc[0] + acc[2]
            outv[s, pl.ds(h * C + 16, 16)] = acc[1] + acc[3]
            return 0

        lax.fori_loop(0, NHEADS, h_body, 0)
        out_copy(pt, s).start()

    idx_copy(base, 0).start()
    wts_copy(base, 0).start()
    wts_copy(base + 1, 1).start()
    idx_copy(base, 0).wait()
    start(gathers(0))
    idx_copy(base + 1, 1).start()

    def pair_body(k, _):
        a = base + 2 * k
        idx_copy(a + 1, 1).wait()
        start(gathers(1))
        wait(gathers(0))
        idx_copy(a + 2, 0).start()
        wts_copy(a, 0).wait()
        compute(a, 0, k)
        wts_copy(a + 2, 0).start()
        idx_copy(a + 2, 0).wait()
        start(gathers(0))
        wait(gathers(1))
        idx_copy(a + 3, 1).start()
        wts_copy(a + 1, 1).wait()
        compute(a + 1, 1, k)
        wts_copy(a + 3, 1).start()
        return 0

    lax.fori_loop(0, PTS_PER_W // 2, pair_body, 0)
    last = base + PTS_PER_W - 1
    wait(gathers(0))
    idx_copy(last + 2, 1).wait()
    wts_copy(last + 1, 0).wait()
    wts_copy(last + 2, 1).wait()
    out_copy(last - 1, 0).wait()
    out_copy(last, 1).wait()


def _sc_gather(table, idx, wts):
    mesh = plsc.VectorSubcoreMesh(core_axis_name="c", subcore_axis_name="s")
    f = functools.partial(
        pl.kernel,
        mesh=mesh,
        compiler_params=pltpu.CompilerParams(use_tc_tiling_on_sc=False,
                                             needs_layout_passes=False),
        out_type=jax.ShapeDtypeStruct((NPTS, HID), jnp.float32),
        scratch_types=[
            pltpu.VMEM((2, 8, 128), jnp.int32),
            pltpu.VMEM((2, NROWS), jnp.float32),
            pltpu.VMEM((2, NROWS // 2, 2 * C), jnp.bfloat16),
            pltpu.VMEM((2, HID), jnp.float32),
            pltpu.SemaphoreType.DMA,
            pltpu.SemaphoreType.DMA,
            pltpu.SemaphoreType.DMA,
            pltpu.SemaphoreType.DMA,
            pltpu.SemaphoreType.DMA,
            pltpu.SemaphoreType.DMA,
            pltpu.SemaphoreType.DMA,
        ],
    )(_sc_body)
    return f(table, idx, wts)


def _proj_body(x_ref, wo_ref, bo_ref, out_ref):
    out_ref[...] = (jnp.dot(x_ref[...], wo_ref[...],
                            preferred_element_type=jnp.float32)
                    + bo_ref[0][None, :])


_kk = np.tile(np.arange(C), NHEADS)
_hh = np.repeat(np.arange(NHEADS), C) * C
_PERM = (_hh + np.where(_kk < 16, 2 * _kk, 2 * (_kk - 16) + 1)).astype(np.int32)


def _out_proj(acc, Wo, bo):
    MB = 256
    return pl.pallas_call(
        _proj_body,
        grid=(NPTS // MB,),
        in_specs=[
            pl.BlockSpec((MB, HID), lambda i: (i, 0)),
            pl.BlockSpec((HID, EMB), lambda i: (0, 0)),
            pl.BlockSpec((1, EMB), lambda i: (0, 0)),
        ],
        out_specs=pl.BlockSpec((MB, EMB), lambda i: (i, 0)),
        out_shape=jax.ShapeDtypeStruct((NPTS, EMB), jnp.float32),
    )(acc, Wo[jnp.asarray(_PERM)], bo.reshape(1, EMB))


def kernel(img, shapes, queries, reference_points, Wi, bi, Wq, bq, Wo, bo):
    table = _imgp_table(img, Wi, bi)
    idx, wts = _points(queries, reference_points, Wq, bq)
    acc = _sc_gather(table, idx, wts)
    out = _out_proj(acc, Wo, bo)
    return out.reshape(B, NQ, EMB)


# speedup vs baseline: 1.2239x; 1.0187x over previous
import functools

import jax
import jax.numpy as jnp
import numpy as np
from jax import lax
from jax.experimental import pallas as pl
from jax.experimental.pallas import tpu as pltpu
from jax.experimental.pallas import tpu_sc as plsc

EMB = 512
HID = 512
NHEADS = 16
NLEVELS = 4
NPOINTS = 8
B = 2
NQ = 1024
LEVEL_SHAPES = [[64, 64], [32, 32], [16, 16], [8, 8]]
I_TOTAL = sum(h * w for h, w in LEVEL_SHAPES)
C = HID // NHEADS
LANES = NHEADS * NLEVELS * NPOINTS
NPTS = B * NQ
NROWS = 4 * LANES
NWORK = 32
PTS_PER_W = NPTS // NWORK

_shapes_np = np.array(LEVEL_SHAPES, np.int32)
_sizes = _shapes_np[:, 0] * _shapes_np[:, 1]
_lev_start = np.concatenate([[0], np.cumsum(_sizes)[:-1]]).astype(np.int32)
_lane_l = (np.arange(LANES) // NPOINTS) % NLEVELS
_lane_h = np.arange(LANES) // (NLEVELS * NPOINTS)
_WM1 = (_shapes_np[_lane_l, 1] - 1).astype(np.float32)
_HM1 = (_shapes_np[_lane_l, 0] - 1).astype(np.float32)
_WVEC = _shapes_np[_lane_l, 1].astype(np.int32)
_LSTART = _lev_start[_lane_l].astype(np.int32)
IBLK = 544
NBLK = I_TOTAL // IBLK
_H544 = (_lane_h * IBLK).astype(np.int32)


def _imgp_body(img_ref, imgn_ref, wi_ref, bi_ref, out_ref):
    x = img_ref[0]
    xn = imgn_ref[0]
    y = jnp.dot(x, wi_ref[...], preferred_element_type=jnp.float32)
    y = y + bi_ref[0][None, :]
    yb = y.astype(jnp.bfloat16)
    yn = jnp.dot(xn, wi_ref[...], preferred_element_type=jnp.float32)
    yn = yn + bi_ref[0][None, :]
    ybn = yn.astype(jnp.bfloat16)
    ysh = jnp.concatenate([yb[1:], ybn[:1]], axis=0)
    IB = yb.shape[0]
    for h in range(NHEADS):
        out_ref[pl.ds(h * IB, IB), :] = jnp.concatenate(
            [yb[:, h * C:(h + 1) * C], ysh[:, h * C:(h + 1) * C]], axis=1)


def _imgp_table(img, Wi, bi):
    grid = (B, NBLK)
    nblk8 = I_TOTAL // 8 - 1
    return pl.pallas_call(
        _imgp_body,
        grid=grid,
        in_specs=[
            pl.BlockSpec((1, IBLK, EMB), lambda b, i: (b, i, 0)),
            pl.BlockSpec((1, 8, EMB),
                         lambda b, i: (b, jnp.minimum((i + 1) * (IBLK // 8), nblk8), 0)),
            pl.BlockSpec((EMB, HID), lambda b, i: (0, 0)),
            pl.BlockSpec((1, HID), lambda b, i: (0, 0)),
        ],
        out_specs=pl.BlockSpec((NHEADS * IBLK, 2 * C),
                               lambda b, i: (b * NBLK + i, 0)),
        out_shape=jax.ShapeDtypeStruct((B * NBLK * NHEADS * IBLK, 2 * C),
                                       jnp.bfloat16),
    )(img, img, Wi, bi.reshape(1, HID))


def _points_body(q_ref, rpx_ref, rpy_ref, wx_ref, wy_ref, wl_ref, bx_ref,
                 by_ref, bl_ref, wm1_ref, hm1_ref, wvec_ref, lstart_ref,
                 h544_ref, idx_ref, wts_ref):
    b = pl.program_id(0)
    q = q_ref[0]
    rpx_row = rpx_ref[0, 0]
    rpy_row = rpy_ref[0, 0]
    ox = jnp.dot(q, wx_ref[...], preferred_element_type=jnp.float32) + bx_ref[0][None, :]
    oy = jnp.dot(q, wy_ref[...], preferred_element_type=jnp.float32) + by_ref[0][None, :]
    lg = jnp.dot(q, wl_ref[...], preferred_element_type=jnp.float32) + bl_ref[0][None, :]
    QB = ox.shape[0]
    lg3 = lg.reshape(QB, NHEADS, NLEVELS * NPOINTS)
    m = jnp.max(lg3, axis=2, keepdims=True)
    e = jnp.exp(lg3 - m)
    aw = (e / jnp.sum(e, axis=2, keepdims=True)).reshape(QB, LANES)

    wm1 = wm1_ref[0][None, :]
    hm1 = hm1_ref[0][None, :]
    spx = rpx_row[:, None] + ox
    spy = rpy_row[:, None] + oy
    x = jnp.clip(spx * wm1, 0.0, wm1)
    y = jnp.clip(spy * hm1, 0.0, hm1)
    x0f = jnp.minimum(jnp.floor(x), wm1 - 1.0)
    y0f = jnp.minimum(jnp.floor(y), hm1 - 1.0)
    fx = x - x0f
    fy = y - y0f
    x0 = x0f.astype(jnp.int32)
    y0 = y0f.astype(jnp.int32)
    wvec = wvec_ref[0][None, :]
    p0 = lstart_ref[0][None, :] + y0 * wvec + x0
    p1 = p0 + wvec
    bbase = b * (NBLK * NHEADS * IBLK) + h544_ref[0][None, :]

    def rowid(p):
        pf = (p.astype(jnp.float32) + 0.5) * jnp.float32(1.0 / IBLK)
        blk = jnp.floor(pf).astype(jnp.int32)
        u = p - blk * IBLK
        return bbase + blk * (NHEADS * IBLK) + u

    i0 = rowid(p0)
    i1 = rowid(p1)
    gx = 1.0 - fx
    gy = 1.0 - fy
    w00 = aw * gy * gx
    w01 = aw * gy * fx
    w10 = aw * fy * gx
    w11 = aw * fy * fx
    for t, iv in enumerate((i0, i1)):
        for k in range(4):
            idx_ref[:, t * 4 + k, :] = iv[:, k * 128:(k + 1) * 128]
    for t, wv in enumerate((w00, w01, w10, w11)):
        wts_ref[:, pl.ds(t * LANES, LANES)] = wv


def _points(queries, reference_points, Wq, bq):
    QB = 128
    Wq3 = Wq.reshape(EMB, LANES, 3)
    Wx = Wq3[..., 0]
    Wy = Wq3[..., 1]
    Wl = Wq3[..., 2]
    bq3 = bq.reshape(LANES, 3)
    bx = bq3[:, 0].reshape(1, LANES)
    by = bq3[:, 1].reshape(1, LANES)
    bl = bq3[:, 2].reshape(1, LANES)
    rpx = reference_points[..., 0].reshape(B * (NQ // QB), 1, QB)
    rpy = reference_points[..., 1].reshape(B * (NQ // QB), 1, QB)
    grid = (B, NQ // QB)
    full = lambda b, i: (0, 0)
    idx, wts = pl.pallas_call(
        _points_body,
        grid=grid,
        in_specs=[
            pl.BlockSpec((1, QB, EMB), lambda b, i: (b, i, 0)),
            pl.BlockSpec((1, 1, QB), lambda b, i: (b * (NQ // QB) + i, 0, 0)),
            pl.BlockSpec((1, 1, QB), lambda b, i: (b * (NQ // QB) + i, 0, 0)),
            pl.BlockSpec((EMB, LANES), full),
            pl.BlockSpec((EMB, LANES), full),
            pl.BlockSpec((EMB, LANES), full),
            pl.BlockSpec((1, LANES), full),
            pl.BlockSpec((1, LANES), full),
            pl.BlockSpec((1, LANES), full),
            pl.BlockSpec((1, LANES), full),
            pl.BlockSpec((1, LANES), full),
            pl.BlockSpec((1, LANES), full),
            pl.BlockSpec((1, LANES), full),
            pl.BlockSpec((1, LANES), full),
        ],
        out_specs=[
            pl.BlockSpec((QB, 8, 128), lambda b, i: (b * (NQ // QB) + i, 0, 0)),
            pl.BlockSpec((QB, NROWS), lambda b, i: (b * (NQ // QB) + i, 0)),
        ],
        out_shape=[
            jax.ShapeDtypeStruct((NPTS, 8, 128), jnp.int32),
            jax.ShapeDtypeStruct((NPTS, NROWS), jnp.float32),
        ],
    )(queries, rpx, rpy, Wx, Wy, Wl, bx, by, bl,
      jnp.asarray(_WM1).reshape(1, LANES), jnp.asarray(_HM1).reshape(1, LANES),
      jnp.asarray(_WVEC).reshape(1, LANES), jnp.asarray(_LSTART).reshape(1, LANES),
      jnp.asarray(_H544).reshape(1, LANES))
    return idx, wts


def _sc_body(table_hbm, idx_hbm, wts_hbm, out_hbm, idxv, wtsv, rowsv, outv,
             sem_rows0, sem_rows1, sem_idx, sem_wts0, sem_wts1,
             sem_out0, sem_out1):
    wid = lax.axis_index("s") * 2 + lax.axis_index("c")
    base = wid * PTS_PER_W
    sem_rows = (sem_rows0, sem_rows1)
    sem_wts = (sem_wts0, sem_wts1)
    sem_out = (sem_out0, sem_out1)

    def clamp(pt):
        return jnp.minimum(pt, NPTS - 1)

    def idx_copy(pt, s):
        return pltpu.make_async_copy(idx_hbm.at[clamp(pt)], idxv.at[s], sem_idx)

    def wts_copy(pt, s):
        return pltpu.make_async_copy(wts_hbm.at[clamp(pt)], wtsv.at[s], sem_wts[s])

    def gathers(s):
        return [
            pltpu.make_async_copy(
                table_hbm.at[idxv.at[s, j]],
                rowsv.at[s, pl.ds(j * 128, 128), :],
                sem_rows[s],
            )
            for j in range(8)
        ]

    def start(cs):
        for cp in cs:
            cp.start()

    def wait(cs):
        for cp in cs:
            cp.wait()

    def out_copy(pt, s):
        return pltpu.make_async_copy(outv.at[s], out_hbm.at[clamp(pt)], sem_out[s])

    def compute(pt, s, k):
        @pl.when(k > 0)
        def _():
            out_copy(pt - 2, s).wait()

        def h_body(h, _):
            hb = h * (NLEVELS * NPOINTS)
            acc = [jnp.zeros((16,), jnp.float32) for _ in range(4)]
            for t in range(2):
                for g in range(2):
                    sb = t * LANES + hb + g * 16
                    wlv = wtsv[s, pl.ds(2 * t * LANES + hb + g * 16, 16)]
                    wrv = wtsv[s, pl.ds((2 * t + 1) * LANES + hb + g * 16, 16)]
                    for q in range(4):
                        p = []
                        for e in range(4):
                            j = q * 4 + e
                            wl = wlv[j]
                            wr = wrv[j]
                            wsl = jnp.full((16,), wl, jnp.float32)
                            wbl = plsc.pack(wsl, wsl,
                                            format=plsc.PackFormat.INTERLEAVED)
                            wsr = jnp.full((16,), wr, jnp.float32)
                            wbr = plsc.pack(wsr, wsr,
                                            format=plsc.PackFormat.INTERLEAVED)
                            p.append(rowsv[s, sb + j, pl.ds(0, 32)] * wbl
                                     + rowsv[s, sb + j, pl.ds(32, 32)] * wbr)
                        tree = (p[0] + p[1]) + (p[2] + p[3])
                        rev, rod = plsc.unpack(
                            tree, format=plsc.PackFormat.INTERLEAVED)
                        kk = (q & 1) * 2
                        acc[kk] = acc[kk] + rev
                        acc[kk + 1] = acc[kk + 1] + rod
            outv[s, pl.ds(h * C, 16)] = acc[0] + acc[2]
            outv[s, pl.ds(h * C + 16, 16)] = acc[1] + acc[3]
            return 0

        lax.fori_loop(0, NHEADS, h_body, 0)
        out_copy(pt, s).start()

    idx_copy(base, 0).start()
    wts_copy(base, 0).start()
    wts_copy(base + 1, 1).start()
    idx_copy(base, 0).wait()
    start(gathers(0))
    idx_copy(base + 1, 1).start()

    def pair_body(k, _):
        a = base + 2 * k
        idx_copy(a + 1, 1).wait()
        start(gathers(1))
        wait(gathers(0))
        idx_copy(a + 2, 0).start()
        wts_copy(a, 0).wait()
        compute(a, 0, k)
        wts_copy(a + 2, 0).start()
        idx_copy(a + 2, 0).wait()
        start(gathers(0))
        wait(gathers(1))
        idx_copy(a + 3, 1).start()
        wts_copy(a + 1, 1).wait()
        compute(a + 1, 1, k)
        wts_copy(a + 3, 1).start()
        return 0

    lax.fori_loop(0, PTS_PER_W // 2, pair_body, 0)
    last = base + PTS_PER_W - 1
    wait(gathers(0))
    idx_copy(last + 2, 1).wait()
    wts_copy(last + 1, 0).wait()
    wts_copy(last + 2, 1).wait()
    out_copy(last - 1, 0).wait()
    out_copy(last, 1).wait()


def _sc_gather(table, idx, wts):
    mesh = plsc.VectorSubcoreMesh(core_axis_name="c", subcore_axis_name="s")
    f = functools.partial(
        pl.kernel,
        mesh=mesh,
        compiler_params=pltpu.CompilerParams(use_tc_tiling_on_sc=False,
                                             needs_layout_passes=False),
        out_type=jax.ShapeDtypeStruct((NPTS, HID), jnp.float32),
        scratch_types=[
            pltpu.VMEM((2, 8, 128), jnp.int32),
            pltpu.VMEM((2, NROWS), jnp.float32),
            pltpu.VMEM((2, NROWS // 2, 2 * C), jnp.bfloat16),
            pltpu.VMEM((2, HID), jnp.float32),
            pltpu.SemaphoreType.DMA,
            pltpu.SemaphoreType.DMA,
            pltpu.SemaphoreType.DMA,
            pltpu.SemaphoreType.DMA,
            pltpu.SemaphoreType.DMA,
            pltpu.SemaphoreType.DMA,
            pltpu.SemaphoreType.DMA,
        ],
    )(_sc_body)
    return f(table, idx, wts)


def _proj_body(x_ref, wo_ref, bo_ref, out_ref):
    out_ref[...] = (jnp.dot(x_ref[...], wo_ref[...],
                            preferred_element_type=jnp.float32)
                    + bo_ref[0][None, :])


_kk = np.tile(np.arange(C), NHEADS)
_hh = np.repeat(np.arange(NHEADS), C) * C
_PERM = (_hh + np.where(_kk < 16, 2 * _kk, 2 * (_kk - 16) + 1)).astype(np.int32)


def _out_proj(acc, Wo, bo):
    MB = 256
    return pl.pallas_call(
        _proj_body,
        grid=(NPTS // MB,),
        in_specs=[
            pl.BlockSpec((MB, HID), lambda i: (i, 0)),
            pl.BlockSpec((HID, EMB), lambda i: (0, 0)),
            pl.BlockSpec((1, EMB), lambda i: (0, 0)),
        ],
        out_specs=pl.BlockSpec((MB, EMB), lambda i: (i, 0)),
        out_shape=jax.ShapeDtypeStruct((NPTS, EMB), jnp.float32),
    )(acc, Wo.reshape(NHEADS, 16, 2, EMB).transpose(0, 2, 1, 3).reshape(HID, EMB),
      bo.reshape(1, EMB))


def kernel(img, shapes, queries, reference_points, Wi, bi, Wq, bq, Wo, bo):
    table = _imgp_table(img, Wi, bi)
    idx, wts = _points(queries, reference_points, Wq, bq)
    acc = _sc_gather(table, idx, wts)
    out = _out_proj(acc, Wo, bo)
    return out.reshape(B, NQ, EMB)


# speedup vs baseline: 1.2742x; 1.0410x over previous
import functools

import jax
import jax.numpy as jnp
import numpy as np
from jax import lax
from jax.experimental import pallas as pl
from jax.experimental.pallas import tpu as pltpu
from jax.experimental.pallas import tpu_sc as plsc

EMB = 512
HID = 512
NHEADS = 16
NLEVELS = 4
NPOINTS = 8
B = 2
NQ = 1024
LEVEL_SHAPES = [[64, 64], [32, 32], [16, 16], [8, 8]]
I_TOTAL = sum(h * w for h, w in LEVEL_SHAPES)
C = HID // NHEADS
LANES = NHEADS * NLEVELS * NPOINTS
NPTS = B * NQ
NROWS = 4 * LANES
NWORK = 32
PTS_PER_W = NPTS // NWORK

_shapes_np = np.array(LEVEL_SHAPES, np.int32)
_sizes = _shapes_np[:, 0] * _shapes_np[:, 1]
_lev_start = np.concatenate([[0], np.cumsum(_sizes)[:-1]]).astype(np.int32)
_lane_l = (np.arange(LANES) // NPOINTS) % NLEVELS
_lane_h = np.arange(LANES) // (NLEVELS * NPOINTS)
_WM1 = (_shapes_np[_lane_l, 1] - 1).astype(np.float32)
_HM1 = (_shapes_np[_lane_l, 0] - 1).astype(np.float32)
_WVEC = _shapes_np[_lane_l, 1].astype(np.int32)
_LSTART = _lev_start[_lane_l].astype(np.int32)
IBLK = 544
NBLK = I_TOTAL // IBLK
_H544 = (_lane_h * IBLK).astype(np.int32)
_SEG = (np.arange(LANES)[:, None] // 32 == np.arange(LANES)[None, :] // 32
        ).astype(np.float32)


def _imgp_body(img_ref, imgn_ref, wi_ref, bi_ref, out_ref):
    x = img_ref[0]
    xn = imgn_ref[0]
    y = jnp.dot(x, wi_ref[...], preferred_element_type=jnp.float32)
    y = y + bi_ref[0][None, :]
    yb = y.astype(jnp.bfloat16)
    yn = jnp.dot(xn, wi_ref[...], preferred_element_type=jnp.float32)
    yn = yn + bi_ref[0][None, :]
    ybn = yn.astype(jnp.bfloat16)
    ysh = jnp.concatenate([yb[1:], ybn[:1]], axis=0)
    IB = yb.shape[0]
    for h in range(NHEADS):
        out_ref[pl.ds(h * IB, IB), :] = jnp.concatenate(
            [yb[:, h * C:(h + 1) * C], ysh[:, h * C:(h + 1) * C]], axis=1)


def _imgp_table(img, Wi, bi):
    grid = (B, NBLK)
    nblk8 = I_TOTAL // 8 - 1
    return pl.pallas_call(
        _imgp_body,
        grid=grid,
        in_specs=[
            pl.BlockSpec((1, IBLK, EMB), lambda b, i: (b, i, 0)),
            pl.BlockSpec((1, 8, EMB),
                         lambda b, i: (b, jnp.minimum((i + 1) * (IBLK // 8), nblk8), 0)),
            pl.BlockSpec((EMB, HID), lambda b, i: (0, 0)),
            pl.BlockSpec((1, HID), lambda b, i: (0, 0)),
        ],
        out_specs=pl.BlockSpec((NHEADS * IBLK, 2 * C),
                               lambda b, i: (b * NBLK + i, 0)),
        out_shape=jax.ShapeDtypeStruct((B * NBLK * NHEADS * IBLK, 2 * C),
                                       jnp.bfloat16),
    )(img, img, Wi, bi.reshape(1, HID))


def _points_body(q_ref, rpx_ref, rpy_ref, wx_ref, wy_ref, wl_ref, bx_ref,
                 by_ref, bl_ref, wm1_ref, hm1_ref, wvec_ref, lstart_ref,
                 h544_ref, seg_ref, idx_ref, wts_ref):
    b = pl.program_id(0)
    q = q_ref[0]
    rpx_row = rpx_ref[0, 0]
    rpy_row = rpy_ref[0, 0]
    ox = jnp.dot(q, wx_ref[...], preferred_element_type=jnp.float32) + bx_ref[0][None, :]
    oy = jnp.dot(q, wy_ref[...], preferred_element_type=jnp.float32) + by_ref[0][None, :]
    lg = jnp.dot(q, wl_ref[...], preferred_element_type=jnp.float32) + bl_ref[0][None, :]
    QB = ox.shape[0]
    m = jnp.max(lg, axis=1, keepdims=True)
    e = jnp.exp(lg - m)
    ssum = jnp.dot(e, seg_ref[...], preferred_element_type=jnp.float32)
    aw = e / ssum

    wm1 = wm1_ref[0][None, :]
    hm1 = hm1_ref[0][None, :]
    spx = rpx_row[:, None] + ox
    spy = rpy_row[:, None] + oy
    x = jnp.clip(spx * wm1, 0.0, wm1)
    y = jnp.clip(spy * hm1, 0.0, hm1)
    x0f = jnp.minimum(jnp.floor(x), wm1 - 1.0)
    y0f = jnp.minimum(jnp.floor(y), hm1 - 1.0)
    fx = x - x0f
    fy = y - y0f
    x0 = x0f.astype(jnp.int32)
    y0 = y0f.astype(jnp.int32)
    wvec = wvec_ref[0][None, :]
    p0 = lstart_ref[0][None, :] + y0 * wvec + x0
    p1 = p0 + wvec
    bbase = b * (NBLK * NHEADS * IBLK) + h544_ref[0][None, :]

    def rowid(p):
        pf = (p.astype(jnp.float32) + 0.5) * jnp.float32(1.0 / IBLK)
        blk = jnp.floor(pf).astype(jnp.int32)
        u = p - blk * IBLK
        return bbase + blk * (NHEADS * IBLK) + u

    i0 = rowid(p0)
    i1 = rowid(p1)
    gx = 1.0 - fx
    gy = 1.0 - fy
    w00 = aw * gy * gx
    w01 = aw * gy * fx
    w10 = aw * fy * gx
    w11 = aw * fy * fx
    for t, iv in enumerate((i0, i1)):
        for k in range(4):
            idx_ref[:, t * 4 + k, :] = iv[:, k * 128:(k + 1) * 128]
    for t, wv in enumerate((w00, w01, w10, w11)):
        wts_ref[:, pl.ds(t * LANES, LANES)] = wv


def _points(queries, reference_points, Wq, bq):
    QB = 128
    Wq3 = Wq.reshape(EMB, LANES, 3)
    Wx = Wq3[..., 0]
    Wy = Wq3[..., 1]
    Wl = Wq3[..., 2]
    bq3 = bq.reshape(LANES, 3)
    bx = bq3[:, 0].reshape(1, LANES)
    by = bq3[:, 1].reshape(1, LANES)
    bl = bq3[:, 2].reshape(1, LANES)
    rpx = reference_points[..., 0].reshape(B * (NQ // QB), 1, QB)
    rpy = reference_points[..., 1].reshape(B * (NQ // QB), 1, QB)
    grid = (B, NQ // QB)
    full = lambda b, i: (0, 0)
    idx, wts = pl.pallas_call(
        _points_body,
        grid=grid,
        in_specs=[
            pl.BlockSpec((1, QB, EMB), lambda b, i: (b, i, 0)),
            pl.BlockSpec((1, 1, QB), lambda b, i: (b * (NQ // QB) + i, 0, 0)),
            pl.BlockSpec((1, 1, QB), lambda b, i: (b * (NQ // QB) + i, 0, 0)),
            pl.BlockSpec((EMB, LANES), full),
            pl.BlockSpec((EMB, LANES), full),
            pl.BlockSpec((EMB, LANES), full),
            pl.BlockSpec((1, LANES), full),
            pl.BlockSpec((1, LANES), full),
            pl.BlockSpec((1, LANES), full),
            pl.BlockSpec((1, LANES), full),
            pl.BlockSpec((1, LANES), full),
            pl.BlockSpec((1, LANES), full),
            pl.BlockSpec((1, LANES), full),
            pl.BlockSpec((1, LANES), full),
            pl.BlockSpec((LANES, LANES), full),
        ],
        out_specs=[
            pl.BlockSpec((QB, 8, 128), lambda b, i: (b * (NQ // QB) + i, 0, 0)),
            pl.BlockSpec((QB, NROWS), lambda b, i: (b * (NQ // QB) + i, 0)),
        ],
        out_shape=[
            jax.ShapeDtypeStruct((NPTS, 8, 128), jnp.int32),
            jax.ShapeDtypeStruct((NPTS, NROWS), jnp.float32),
        ],
    )(queries, rpx, rpy, Wx, Wy, Wl, bx, by, bl,
      jnp.asarray(_WM1).reshape(1, LANES), jnp.asarray(_HM1).reshape(1, LANES),
      jnp.asarray(_WVEC).reshape(1, LANES), jnp.asarray(_LSTART).reshape(1, LANES),
      jnp.asarray(_H544).reshape(1, LANES), jnp.asarray(_SEG))
    return idx, wts


def _sc_body(table_hbm, idx_hbm, wts_hbm, out_hbm, idxv, wtsv, rowsv, outv,
             sem_rows0, sem_rows1, sem_idx, sem_wts0, sem_wts1,
             sem_out0, sem_out1):
    wid = lax.axis_index("s") * 2 + lax.axis_index("c")
    base = wid * PTS_PER_W
    sem_rows = (sem_rows0, sem_rows1)
    sem_wts = (sem_wts0, sem_wts1)
    sem_out = (sem_out0, sem_out1)

    def clamp(pt):
        return jnp.minimum(pt, NPTS - 1)

    def idx_copy(pt, s):
        return pltpu.make_async_copy(idx_hbm.at[clamp(pt)], idxv.at[s], sem_idx)

    def wts_copy(pt, s):
        return pltpu.make_async_copy(wts_hbm.at[clamp(pt)], wtsv.at[s], sem_wts[s])

    def gathers(s):
        return [
            pltpu.make_async_copy(
                table_hbm.at[idxv.at[s, j]],
                rowsv.at[s, pl.ds(j * 128, 128), :],
                sem_rows[s],
            )
            for j in range(8)
        ]

    def start(cs):
        for cp in cs:
            cp.start()

    def wait(cs):
        for cp in cs:
            cp.wait()

    def out_copy(pt, s):
        return pltpu.make_async_copy(outv.at[s], out_hbm.at[clamp(pt)], sem_out[s])

    def compute(pt, s, k):
        @pl.when(k > 0)
        def _():
            out_copy(pt - 2, s).wait()

        def h_body(h, _):
            hb = h * (NLEVELS * NPOINTS)
            acc = [jnp.zeros((16,), jnp.float32) for _ in range(4)]
            for t in range(2):
                for g in range(2):
                    sb = t * LANES + hb + g * 16
                    wlv = wtsv[s, pl.ds(2 * t * LANES + hb + g * 16, 16)]
                    wrv = wtsv[s, pl.ds((2 * t + 1) * LANES + hb + g * 16, 16)]
                    for q in range(4):
                        p = []
                        for e in range(4):
                            j = q * 4 + e
                            wl = wlv[j]
                            wr = wrv[j]
                            wsl = jnp.full((16,), wl, jnp.float32)
                            wbl = plsc.pack(wsl, wsl,
                                            format=plsc.PackFormat.INTERLEAVED)
                            wsr = jnp.full((16,), wr, jnp.float32)
                            wbr = plsc.pack(wsr, wsr,
                                            format=plsc.PackFormat.INTERLEAVED)
                            p.append(rowsv[s, sb + j, pl.ds(0, 32)] * wbl
                                     + rowsv[s, sb + j, pl.ds(32, 32)] * wbr)
                        tree = (p[0] + p[1]) + (p[2] + p[3])
                        rev, rod = plsc.unpack(
                            tree, format=plsc.PackFormat.INTERLEAVED)
                        kk = (q & 1) * 2
                        acc[kk] = acc[kk] + rev
                        acc[kk + 1] = acc[kk + 1] + rod
            outv[s, pl.ds(h * C, 16)] = acc[0] + acc[2]
            outv[s, pl.ds(h * C + 16, 16)] = acc[1] + acc[3]
            return 0

        lax.fori_loop(0, NHEADS, h_body, 0)
        out_copy(pt, s).start()

    idx_copy(base, 0).start()
    wts_copy(base, 0).start()
    wts_copy(base + 1, 1).start()
    idx_copy(base, 0).wait()
    start(gathers(0))
    idx_copy(base + 1, 1).start()

    def pair_body(k, _):
        a = base + 2 * k
        idx_copy(a + 1, 1).wait()
        start(gathers(1))
        wait(gathers(0))
        idx_copy(a + 2, 0).start()
        wts_copy(a, 0).wait()
        compute(a, 0, k)
        wts_copy(a + 2, 0).start()
        idx_copy(a + 2, 0).wait()
        start(gathers(0))
        wait(gathers(1))
        idx_copy(a + 3, 1).start()
        wts_copy(a + 1, 1).wait()
        compute(a + 1, 1, k)
        wts_copy(a + 3, 1).start()
        return 0

    lax.fori_loop(0, PTS_PER_W // 2, pair_body, 0)
    last = base + PTS_PER_W - 1
    wait(gathers(0))
    idx_copy(last + 2, 1).wait()
    wts_copy(last + 1, 0).wait()
    wts_copy(last + 2, 1).wait()
    out_copy(last - 1, 0).wait()
    out_copy(last, 1).wait()


def _sc_gather(table, idx, wts):
    mesh = plsc.VectorSubcoreMesh(core_axis_name="c", subcore_axis_name="s")
    f = functools.partial(
        pl.kernel,
        mesh=mesh,
        compiler_params=pltpu.CompilerParams(use_tc_tiling_on_sc=False,
                                             needs_layout_passes=False),
        out_type=jax.ShapeDtypeStruct((NPTS, HID), jnp.float32),
        scratch_types=[
            pltpu.VMEM((2, 8, 128), jnp.int32),
            pltpu.VMEM((2, NROWS), jnp.float32),
            pltpu.VMEM((2, NROWS // 2, 2 * C), jnp.bfloat16),
            pltpu.VMEM((2, HID), jnp.float32),
            pltpu.SemaphoreType.DMA,
            pltpu.SemaphoreType.DMA,
            pltpu.SemaphoreType.DMA,
            pltpu.SemaphoreType.DMA,
            pltpu.SemaphoreType.DMA,
            pltpu.SemaphoreType.DMA,
            pltpu.SemaphoreType.DMA,
        ],
    )(_sc_body)
    return f(table, idx, wts)


def _proj_body(x_ref, wo_ref, bo_ref, out_ref):
    out_ref[...] = (jnp.dot(x_ref[...], wo_ref[...],
                            preferred_element_type=jnp.float32)
                    + bo_ref[0][None, :])


_kk = np.tile(np.arange(C), NHEADS)
_hh = np.repeat(np.arange(NHEADS), C) * C
_PERM = (_hh + np.where(_kk < 16, 2 * _kk, 2 * (_kk - 16) + 1)).astype(np.int32)


def _out_proj(acc, Wo, bo):
    MB = 256
    return pl.pallas_call(
        _proj_body,
        grid=(NPTS // MB,),
        in_specs=[
            pl.BlockSpec((MB, HID), lambda i: (i, 0)),
            pl.BlockSpec((HID, EMB), lambda i: (0, 0)),
            pl.BlockSpec((1, EMB), lambda i: (0, 0)),
        ],
        out_specs=pl.BlockSpec((MB, EMB), lambda i: (i, 0)),
        out_shape=jax.ShapeDtypeStruct((NPTS, EMB), jnp.float32),
    )(acc, Wo.reshape(NHEADS, 16, 2, EMB).transpose(0, 2, 1, 3).reshape(HID, EMB),
      bo.reshape(1, EMB))


def kernel(img, shapes, queries, reference_points, Wi, bi, Wq, bq, Wo, bo):
    table = _imgp_table(img, Wi, bi)
    idx, wts = _points(queries, reference_points, Wq, bq)
    acc = _sc_gather(table, idx, wts)
    out = _out_proj(acc, Wo, bo)
    return out.reshape(B, NQ, EMB)


# speedup vs baseline: 1.2770x; 1.0022x over previous
import functools

import jax
import jax.numpy as jnp
import numpy as np
from jax import lax
from jax.experimental import pallas as pl
from jax.experimental.pallas import tpu as pltpu
from jax.experimental.pallas import tpu_sc as plsc

EMB = 512
HID = 512
NHEADS = 16
NLEVELS = 4
NPOINTS = 8
B = 2
NQ = 1024
LEVEL_SHAPES = [[64, 64], [32, 32], [16, 16], [8, 8]]
I_TOTAL = sum(h * w for h, w in LEVEL_SHAPES)
C = HID // NHEADS
LANES = NHEADS * NLEVELS * NPOINTS
NPTS = B * NQ
NROWS = 4 * LANES
NWORK = 32
PTS_PER_W = NPTS // NWORK

_shapes_np = np.array(LEVEL_SHAPES, np.int32)
_sizes = _shapes_np[:, 0] * _shapes_np[:, 1]
_lev_start = np.concatenate([[0], np.cumsum(_sizes)[:-1]]).astype(np.int32)
_lane_l = (np.arange(LANES) // NPOINTS) % NLEVELS
_lane_h = np.arange(LANES) // (NLEVELS * NPOINTS)
_WM1 = (_shapes_np[_lane_l, 1] - 1).astype(np.float32)
_HM1 = (_shapes_np[_lane_l, 0] - 1).astype(np.float32)
_WVEC = _shapes_np[_lane_l, 1].astype(np.int32)
_LSTART = _lev_start[_lane_l].astype(np.int32)
IBLK = 544
NBLK = I_TOTAL // IBLK
_H544 = (_lane_h * IBLK).astype(np.int32)
_SEG = (np.arange(LANES)[:, None] // 32 == np.arange(LANES)[None, :] // 32
        ).astype(np.float32)


def _imgp_body(img_ref, imgn_ref, wi_ref, bi_ref, out_ref):
    x = img_ref[0]
    xn = imgn_ref[0]
    y = jnp.dot(x, wi_ref[...], preferred_element_type=jnp.float32)
    y = y + bi_ref[0][None, :]
    yb = y.astype(jnp.bfloat16)
    yn = jnp.dot(xn, wi_ref[...], preferred_element_type=jnp.float32)
    yn = yn + bi_ref[0][None, :]
    ybn = yn.astype(jnp.bfloat16)
    ysh = jnp.concatenate([yb[1:], ybn[:1]], axis=0)
    IB = yb.shape[0]
    for h in range(NHEADS):
        out_ref[pl.ds(h * IB, IB), :] = jnp.concatenate(
            [yb[:, h * C:(h + 1) * C], ysh[:, h * C:(h + 1) * C]], axis=1)


def _imgp_table(img, Wi, bi):
    grid = (B, NBLK)
    nblk8 = I_TOTAL // 8 - 1
    return pl.pallas_call(
        _imgp_body,
        grid=grid,
        in_specs=[
            pl.BlockSpec((1, IBLK, EMB), lambda b, i: (b, i, 0)),
            pl.BlockSpec((1, 8, EMB),
                         lambda b, i: (b, jnp.minimum((i + 1) * (IBLK // 8), nblk8), 0)),
            pl.BlockSpec((EMB, HID), lambda b, i: (0, 0)),
            pl.BlockSpec((1, HID), lambda b, i: (0, 0)),
        ],
        out_specs=pl.BlockSpec((NHEADS * IBLK, 2 * C),
                               lambda b, i: (b * NBLK + i, 0)),
        out_shape=jax.ShapeDtypeStruct((B * NBLK * NHEADS * IBLK, 2 * C),
                                       jnp.bfloat16),
    )(img, img, Wi, bi.reshape(1, HID))


def _points_body(q_ref, rpx_ref, rpy_ref, wx_ref, wy_ref, wl_ref, bx_ref,
                 by_ref, bl_ref, wm1_ref, hm1_ref, wvec_ref, lstart_ref,
                 h544_ref, seg_ref, idx_ref, wts_ref):
    b = pl.program_id(0)
    q = q_ref[0]
    rpx_row = rpx_ref[0, 0]
    rpy_row = rpy_ref[0, 0]
    ox = jnp.dot(q, wx_ref[...], preferred_element_type=jnp.float32) + bx_ref[0][None, :]
    oy = jnp.dot(q, wy_ref[...], preferred_element_type=jnp.float32) + by_ref[0][None, :]
    lg = jnp.dot(q, wl_ref[...], preferred_element_type=jnp.float32) + bl_ref[0][None, :]
    QB = ox.shape[0]
    m = jnp.max(lg, axis=1, keepdims=True)
    e = jnp.exp(lg - m)
    ssum = jnp.dot(e, seg_ref[...], preferred_element_type=jnp.float32)
    aw = e / ssum

    wm1 = wm1_ref[0][None, :]
    hm1 = hm1_ref[0][None, :]
    spx = rpx_row[:, None] + ox
    spy = rpy_row[:, None] + oy
    x = jnp.clip(spx * wm1, 0.0, wm1)
    y = jnp.clip(spy * hm1, 0.0, hm1)
    x0f = jnp.minimum(jnp.floor(x), wm1 - 1.0)
    y0f = jnp.minimum(jnp.floor(y), hm1 - 1.0)
    fx = x - x0f
    fy = y - y0f
    x0 = x0f.astype(jnp.int32)
    y0 = y0f.astype(jnp.int32)
    wvec = wvec_ref[0][None, :]
    p0 = lstart_ref[0][None, :] + y0 * wvec + x0
    p1 = p0 + wvec
    bbase = b * (NBLK * NHEADS * IBLK) + h544_ref[0][None, :]

    def rowid(p):
        pf = (p.astype(jnp.float32) + 0.5) * jnp.float32(1.0 / IBLK)
        blk = jnp.floor(pf).astype(jnp.int32)
        u = p - blk * IBLK
        return bbase + blk * (NHEADS * IBLK) + u

    i0 = rowid(p0)
    i1 = rowid(p1)
    gx = 1.0 - fx
    gy = 1.0 - fy
    w00 = aw * gy * gx
    w01 = aw * gy * fx
    w10 = aw * fy * gx
    w11 = aw * fy * fx
    for t, iv in enumerate((i0, i1)):
        for k in range(4):
            idx_ref[:, t * 4 + k, :] = iv[:, k * 128:(k + 1) * 128]
    for t, wv in enumerate((w00, w01, w10, w11)):
        wts_ref[:, pl.ds(t * LANES, LANES)] = wv


def _points(queries, reference_points, Wq, bq):
    QB = 128
    Wq3 = Wq.reshape(EMB, LANES, 3)
    Wx = Wq3[..., 0]
    Wy = Wq3[..., 1]
    Wl = Wq3[..., 2]
    bq3 = bq.reshape(LANES, 3)
    bx = bq3[:, 0].reshape(1, LANES)
    by = bq3[:, 1].reshape(1, LANES)
    bl = bq3[:, 2].reshape(1, LANES)
    rpx = reference_points[..., 0].reshape(B * (NQ // QB), 1, QB)
    rpy = reference_points[..., 1].reshape(B * (NQ // QB), 1, QB)
    grid = (B, NQ // QB)
    full = lambda b, i: (0, 0)
    idx, wts = pl.pallas_call(
        _points_body,
        grid=grid,
        in_specs=[
            pl.BlockSpec((1, QB, EMB), lambda b, i: (b, i, 0)),
            pl.BlockSpec((1, 1, QB), lambda b, i: (b * (NQ // QB) + i, 0, 0)),
            pl.BlockSpec((1, 1, QB), lambda b, i: (b * (NQ // QB) + i, 0, 0)),
            pl.BlockSpec((EMB, LANES), full),
            pl.BlockSpec((EMB, LANES), full),
            pl.BlockSpec((EMB, LANES), full),
            pl.BlockSpec((1, LANES), full),
            pl.BlockSpec((1, LANES), full),
            pl.BlockSpec((1, LANES), full),
            pl.BlockSpec((1, LANES), full),
            pl.BlockSpec((1, LANES), full),
            pl.BlockSpec((1, LANES), full),
            pl.BlockSpec((1, LANES), full),
            pl.BlockSpec((1, LANES), full),
            pl.BlockSpec((LANES, LANES), full),
        ],
        out_specs=[
            pl.BlockSpec((QB, 8, 128), lambda b, i: (b * (NQ // QB) + i, 0, 0)),
            pl.BlockSpec((QB, NROWS), lambda b, i: (b * (NQ // QB) + i, 0)),
        ],
        out_shape=[
            jax.ShapeDtypeStruct((NPTS, 8, 128), jnp.int32),
            jax.ShapeDtypeStruct((NPTS, NROWS), jnp.float32),
        ],
    )(queries, rpx, rpy, Wx, Wy, Wl, bx, by, bl,
      jnp.asarray(_WM1).reshape(1, LANES), jnp.asarray(_HM1).reshape(1, LANES),
      jnp.asarray(_WVEC).reshape(1, LANES), jnp.asarray(_LSTART).reshape(1, LANES),
      jnp.asarray(_H544).reshape(1, LANES), jnp.asarray(_SEG))
    return idx, wts


def _sc_body(table_hbm, idx_hbm, wts_hbm, out_hbm, idxv, wtsv, rowsv, outv,
             sem_rows0, sem_rows1, sem_idx, sem_wts0, sem_wts1,
             sem_out0, sem_out1):
    wid = lax.axis_index("s") * 2 + lax.axis_index("c")
    base = wid * PTS_PER_W
    sem_rows = (sem_rows0, sem_rows1)
    sem_wts = (sem_wts0, sem_wts1)
    sem_out = (sem_out0, sem_out1)

    def clamp(pt):
        return jnp.minimum(pt, NPTS - 1)

    def idx_copy(pt, s):
        return pltpu.make_async_copy(idx_hbm.at[clamp(pt)], idxv.at[s], sem_idx)

    def wts_copy(pt, s):
        return pltpu.make_async_copy(wts_hbm.at[clamp(pt)], wtsv.at[s], sem_wts[s])

    def gathers(s):
        return [
            pltpu.make_async_copy(
                table_hbm.at[idxv.at[s, j]],
                rowsv.at[s, pl.ds(j * 128, 128), :],
                sem_rows[s],
            )
            for j in range(8)
        ]

    def start(cs):
        for cp in cs:
            cp.start()

    def wait(cs):
        for cp in cs:
            cp.wait()

    def out_copy(pt, s):
        return pltpu.make_async_copy(outv.at[s], out_hbm.at[clamp(pt)], sem_out[s])

    def compute(pt, s, k):
        @pl.when(k > 0)
        def _():
            out_copy(pt - 2, s).wait()

        def h_body(h, _):
            hb = h * (NLEVELS * NPOINTS)
            acc = [jnp.zeros((16,), jnp.float32) for _ in range(4)]
            for t in range(2):
                for g in range(2):
                    sb = t * LANES + hb + g * 16
                    wlv = wtsv[s, pl.ds(2 * t * LANES + hb + g * 16, 16)]
                    wrv = wtsv[s, pl.ds((2 * t + 1) * LANES + hb + g * 16, 16)]
                    for q in range(4):
                        p = []
                        for e in range(4):
                            j = q * 4 + e
                            wl = wlv[j]
                            wr = wrv[j]
                            wsl = jnp.full((16,), wl, jnp.float32)
                            wbl = plsc.pack(wsl, wsl,
                                            format=plsc.PackFormat.INTERLEAVED)
                            wsr = jnp.full((16,), wr, jnp.float32)
                            wbr = plsc.pack(wsr, wsr,
                                            format=plsc.PackFormat.INTERLEAVED)
                            p.append(rowsv[s, sb + j, pl.ds(0, 32)] * wbl
                                     + rowsv[s, sb + j, pl.ds(32, 32)] * wbr)
                        tree = (p[0] + p[1]) + (p[2] + p[3])
                        rev, rod = plsc.unpack(
                            tree, format=plsc.PackFormat.INTERLEAVED)
                        kk = (q & 1) * 2
                        acc[kk] = acc[kk] + rev
                        acc[kk + 1] = acc[kk + 1] + rod
            outv[s, pl.ds(h * C, 16)] = acc[0] + acc[2]
            outv[s, pl.ds(h * C + 16, 16)] = acc[1] + acc[3]
            return 0

        lax.fori_loop(0, NHEADS, h_body, 0)
        out_copy(pt, s).start()

    idx_copy(base, 0).start()
    wts_copy(base, 0).start()
    wts_copy(base + 1, 1).start()
    idx_copy(base, 0).wait()
    start(gathers(0))
    idx_copy(base + 1, 1).start()

    def pair_body(k, _):
        a = base + 2 * k
        idx_copy(a + 1, 1).wait()
        start(gathers(1))
        wait(gathers(0))
        idx_copy(a + 2, 0).start()
        wts_copy(a, 0).wait()
        compute(a, 0, k)
        wts_copy(a + 2, 0).start()
        idx_copy(a + 2, 0).wait()
        start(gathers(0))
        wait(gathers(1))
        idx_copy(a + 3, 1).start()
        wts_copy(a + 1, 1).wait()
        compute(a + 1, 1, k)
        wts_copy(a + 3, 1).start()
        return 0

    lax.fori_loop(0, PTS_PER_W // 2, pair_body, 0)
    last = base + PTS_PER_W - 1
    wait(gathers(0))
    idx_copy(last + 2, 1).wait()
    wts_copy(last + 1, 0).wait()
    wts_copy(last + 2, 1).wait()
    out_copy(last - 1, 0).wait()
    out_copy(last, 1).wait()


def _sc_gather(table, idx, wts):
    mesh = plsc.VectorSubcoreMesh(core_axis_name="c", subcore_axis_name="s")
    f = functools.partial(
        pl.kernel,
        mesh=mesh,
        compiler_params=pltpu.CompilerParams(use_tc_tiling_on_sc=False,
                                             needs_layout_passes=False),
        out_type=jax.ShapeDtypeStruct((NPTS, HID), jnp.float32),
        scratch_types=[
            pltpu.VMEM((2, 8, 128), jnp.int32),
            pltpu.VMEM((2, NROWS), jnp.float32),
            pltpu.VMEM((2, NROWS // 2, 2 * C), jnp.bfloat16),
            pltpu.VMEM((2, HID), jnp.float32),
            pltpu.SemaphoreType.DMA,
            pltpu.SemaphoreType.DMA,
            pltpu.SemaphoreType.DMA,
            pltpu.SemaphoreType.DMA,
            pltpu.SemaphoreType.DMA,
            pltpu.SemaphoreType.DMA,
            pltpu.SemaphoreType.DMA,
        ],
    )(_sc_body)
    return f(table, idx, wts)


def _proj_body(x_ref, wo_ref, bo_ref, out_ref):
    out_ref[...] = (jnp.dot(x_ref[...], wo_ref[...],
                            preferred_element_type=jnp.float32)
                    + bo_ref[0][None, :])


def _out_proj(acc, Wo, bo):
    MB = 256
    return pl.pallas_call(
        _proj_body,
        grid=(NPTS // MB,),
        in_specs=[
            pl.BlockSpec((MB, HID), lambda i: (i, 0)),
            pl.BlockSpec((HID, EMB), lambda i: (0, 0)),
            pl.BlockSpec((1, EMB), lambda i: (0, 0)),
        ],
        out_specs=pl.BlockSpec((MB, EMB), lambda i: (i, 0)),
        out_shape=jax.ShapeDtypeStruct((NPTS, EMB), jnp.float32),
    )(acc, Wo.reshape(NHEADS, 16, 2, EMB).transpose(0, 2, 1, 3).reshape(HID, EMB),
      bo.reshape(1, EMB))


def kernel(img, shapes, queries, reference_points, Wi, bi, Wq, bq, Wo, bo):
    table = _imgp_table(img, Wi, bi)
    idx, wts = _points(queries, reference_points, Wq, bq)
    acc = _sc_gather(table, idx, wts)
    out = _out_proj(acc, Wo, bo)
    return out.reshape(B, NQ, EMB)
